# baseline (speedup 1.0000x reference)
import jax
import jax.numpy as jnp
from jax.experimental import pallas as pl
from jax.experimental.pallas import tpu as pltpu

_OP_EQ, _OP_LT, _OP_GT = 0, 1, 2
_ALPHA, _BETA = 1.2, 0.7
_B, _D = 8192, 2048
_BM = 1024
_NB = _B // _BM
_L = 128
_TR = 64
_NK = _D // _L
_NT = _BM // _TR


def _loss_body(op_ref, j1_ref, j2_ref, out_ref, accn_ref, acc0_ref,
               cnt_ref, pd_ref):
    i = pl.program_id(0)

    @pl.when(i == 0)
    def _init():
        accn_ref[...] = jnp.zeros((2, _TR, _L), jnp.float32)
        acc0_ref[...] = jnp.zeros((1, _L), jnp.float32)
        cnt_ref[...] = jnp.zeros((3, 1, _BM), jnp.float32)

    ones_b = jnp.ones((_L, _L), dtype=jnp.bfloat16)
    acc1 = accn_ref[0]
    acc2 = accn_ref[1]
    for r in range(_NT):
        r0 = r * _TR
        a = j1_ref[r0:r0 + _TR, 0:_L].astype(jnp.bfloat16)
        b = j2_ref[r0:r0 + _TR, 0:_L].astype(jnp.bfloat16)
        p1 = a * a
        p2 = b * b
        p12 = a * b
        for k in range(1, _NK):
            c0 = k * _L
            a = j1_ref[r0:r0 + _TR, c0:c0 + _L].astype(jnp.bfloat16)
            b = j2_ref[r0:r0 + _TR, c0:c0 + _L].astype(jnp.bfloat16)
            p1 += a * a
            p2 += b * b
            p12 += a * b
        s1 = jax.lax.dot(p1, ones_b, preferred_element_type=jnp.float32)
        s2 = jax.lax.dot(p2, ones_b, preferred_element_type=jnp.float32)
        pd_ref[r0:r0 + _TR, :] = p1 + p2 - 2.0 * p12
        dn = jnp.sqrt(s1) - jnp.sqrt(s2)
        acc1 = acc1 + jnp.maximum(dn, 0.0)
        acc2 = acc2 + jnp.maximum(-dn, 0.0)
    accn_ref[0] = acc1
    accn_ref[1] = acc2

    opv = op_ref[...]
    eqv = (opv == _OP_EQ).astype(jnp.float32)
    cnt_ref[0] += eqv
    cnt_ref[1] += (opv == _OP_LT).astype(jnp.float32)
    cnt_ref[2] += (opv == _OP_GT).astype(jnp.float32)
    acc0_ref[...] += jax.lax.dot(eqv.astype(jnp.bfloat16), pd_ref[...],
                                 preferred_element_type=jnp.float32)

    @pl.when(i == _NB - 1)
    def _finalize():
        inv_l = 1.0 / _L
        eq_sd = jnp.sum(acc0_ref[...])
        lt_sum = jnp.sum(accn_ref[0]) * inv_l
        gt_sum = jnp.sum(accn_ref[1]) * inv_l
        eq_cnt = jnp.sum(cnt_ref[0])
        lt_cnt = jnp.sum(cnt_ref[1])
        gt_cnt = jnp.sum(cnt_ref[2])
        has_lt = (lt_cnt > 0.0).astype(jnp.float32)
        has_gt = (gt_cnt > 0.0).astype(jnp.float32)
        eq_loss = eq_sd / jnp.maximum(eq_cnt * _D, 1.0)
        lt_loss = lt_sum * (1.0 / _B)
        gt_loss = gt_sum * (1.0 / _B)
        out_ref[0, 0] = (_ALPHA * eq_loss
                         + _BETA * (has_lt * lt_loss + has_gt * gt_loss))


def kernel(joint1_embedding, joint2_embedding, operation):
    op_row = operation.reshape(1, _B)
    out = pl.pallas_call(
        _loss_body,
        grid=(_NB,),
        in_specs=[
            pl.BlockSpec((1, _BM), lambda i: (0, i)),
            pl.BlockSpec((_BM, _D), lambda i: (i, 0)),
            pl.BlockSpec((_BM, _D), lambda i: (i, 0)),
        ],
        out_specs=pl.BlockSpec(memory_space=pltpu.SMEM),
        out_shape=jax.ShapeDtypeStruct((1, 1), jnp.float32),
        scratch_shapes=[
            pltpu.VMEM((2, _TR, _L), jnp.float32),
            pltpu.VMEM((1, _L), jnp.float32),
            pltpu.VMEM((3, 1, _BM), jnp.float32),
            pltpu.VMEM((_BM, _L), jnp.bfloat16),
        ],
    )(op_row, joint1_embedding, joint2_embedding)
    return out[0, 0]


# speedup vs baseline: 1.0389x; 1.0389x over previous
import jax
import jax.numpy as jnp
from jax.experimental import pallas as pl
from jax.experimental.pallas import tpu as pltpu

_OP_EQ, _OP_LT, _OP_GT = 0, 1, 2
_ALPHA, _BETA = 1.2, 0.7
_B, _D = 8192, 2048
_BM = 512
_NB = _B // _BM
_L = 128
_TR = 64
_NK = _D // _L
_NT = _BM // _TR


def _loss_body(op_ref, j1_ref, j2_ref, out_ref, accn_ref, acc0_ref,
               cnt_ref, pd_ref):
    i = pl.program_id(0)

    @pl.when(i == 0)
    def _init():
        accn_ref[...] = jnp.zeros((2, _TR, _L), jnp.float32)
        acc0_ref[...] = jnp.zeros((1, _L), jnp.float32)
        cnt_ref[...] = jnp.zeros((3, 1, _BM), jnp.float32)

    ones_b = jnp.ones((_L, _L), dtype=jnp.bfloat16)
    acc1 = accn_ref[0]
    acc2 = accn_ref[1]
    for r in range(_NT):
        r0 = r * _TR
        a = j1_ref[r0:r0 + _TR, 0:_L].astype(jnp.bfloat16)
        b = j2_ref[r0:r0 + _TR, 0:_L].astype(jnp.bfloat16)
        p1 = a * a
        p2 = b * b
        p12 = a * b
        for k in range(1, _NK):
            c0 = k * _L
            a = j1_ref[r0:r0 + _TR, c0:c0 + _L].astype(jnp.bfloat16)
            b = j2_ref[r0:r0 + _TR, c0:c0 + _L].astype(jnp.bfloat16)
            p1 += a * a
            p2 += b * b
            p12 += a * b
        s1 = jax.lax.dot(p1, ones_b, preferred_element_type=jnp.float32)
        s2 = jax.lax.dot(p2, ones_b, preferred_element_type=jnp.float32)
        pd_ref[r0:r0 + _TR, :] = p1 + p2 - 2.0 * p12
        dn = jnp.sqrt(s1) - jnp.sqrt(s2)
        acc1 = acc1 + jnp.maximum(dn, 0.0)
        acc2 = acc2 + jnp.maximum(-dn, 0.0)
    accn_ref[0] = acc1
    accn_ref[1] = acc2

    opv = op_ref[...]
    eqv = (opv == _OP_EQ).astype(jnp.float32)
    cnt_ref[0] += eqv
    cnt_ref[1] += (opv == _OP_LT).astype(jnp.float32)
    cnt_ref[2] += (opv == _OP_GT).astype(jnp.float32)
    acc0_ref[...] += jax.lax.dot(eqv.astype(jnp.bfloat16), pd_ref[...],
                                 preferred_element_type=jnp.float32)

    @pl.when(i == _NB - 1)
    def _finalize():
        inv_l = 1.0 / _L
        eq_sd = jnp.sum(acc0_ref[...])
        lt_sum = jnp.sum(accn_ref[0]) * inv_l
        gt_sum = jnp.sum(accn_ref[1]) * inv_l
        eq_cnt = jnp.sum(cnt_ref[0])
        lt_cnt = jnp.sum(cnt_ref[1])
        gt_cnt = jnp.sum(cnt_ref[2])
        has_lt = (lt_cnt > 0.0).astype(jnp.float32)
        has_gt = (gt_cnt > 0.0).astype(jnp.float32)
        eq_loss = eq_sd / jnp.maximum(eq_cnt * _D, 1.0)
        lt_loss = lt_sum * (1.0 / _B)
        gt_loss = gt_sum * (1.0 / _B)
        out_ref[0, 0] = (_ALPHA * eq_loss
                         + _BETA * (has_lt * lt_loss + has_gt * gt_loss))


def kernel(joint1_embedding, joint2_embedding, operation):
    op_row = operation.reshape(1, _B)
    out = pl.pallas_call(
        _loss_body,
        grid=(_NB,),
        in_specs=[
            pl.BlockSpec((1, _BM), lambda i: (0, i)),
            pl.BlockSpec((_BM, _D), lambda i: (i, 0)),
            pl.BlockSpec((_BM, _D), lambda i: (i, 0)),
        ],
        out_specs=pl.BlockSpec(memory_space=pltpu.SMEM),
        out_shape=jax.ShapeDtypeStruct((1, 1), jnp.float32),
        scratch_shapes=[
            pltpu.VMEM((2, _TR, _L), jnp.float32),
            pltpu.VMEM((1, _L), jnp.float32),
            pltpu.VMEM((3, 1, _BM), jnp.float32),
            pltpu.VMEM((_BM, _L), jnp.bfloat16),
        ],
    )(op_row, joint1_embedding, joint2_embedding)
    return out[0, 0]
